# baseline (speedup 1.0000x reference)
import functools

import jax
import jax.numpy as jnp
from jax.experimental import pallas as pl
from jax.experimental.pallas import tpu as pltpu

TOP_K = 2
EXPLORATION_BONUS = 0.1


def _body(x_ref, w1_ref, b1_ref, w2_ref, b2_ref, s1_ref, s1b_ref,
          s2_ref, s2b_ref, we_ref, be_ref, pred_ref, rw_ref, web_scr,
          *, n_experts):
    @pl.when(pl.program_id(0) == 0)
    def _prep():
        web_scr[...] = (we_ref[...].reshape(n_experts, -1).T
                        .astype(jnp.bfloat16))

    xt = x_ref[...]
    xb = xt.astype(jnp.bfloat16)
    h = jnp.maximum(
        jnp.dot(xt, w1_ref[...], preferred_element_type=jnp.float32)
        + b1_ref[...].reshape(1, -1), 0.0)
    ctx = jnp.tanh(
        jnp.dot(h, w2_ref[...], preferred_element_type=jnp.float32)
        + b2_ref[...].reshape(1, -1))
    sh = jnp.maximum(
        jnp.dot(ctx, s1_ref[...], preferred_element_type=jnp.float32)
        + s1b_ref[...].reshape(1, -1), 0.0)
    scores = (jnp.dot(sh, s2_ref[...], preferred_element_type=jnp.float32)
              + s2b_ref[...].reshape(1, -1) + EXPLORATION_BONUS)

    preds = (jnp.dot(xb, web_scr[...], preferred_element_type=jnp.float32)
             + be_ref[...].reshape(1, -1))

    eidx = jax.lax.broadcasted_iota(jnp.int32, scores.shape, 1).astype(
        jnp.float32)
    m1 = jnp.max(scores, axis=1, keepdims=True)
    i1 = jnp.min(jnp.where(scores == m1, eidx, float(n_experts)), axis=1,
                 keepdims=True)
    masked = jnp.where(eidx == i1, -jnp.inf, scores)
    m2 = jnp.max(masked, axis=1, keepdims=True)
    i2 = jnp.min(jnp.where(masked == m2, eidx, float(n_experts)), axis=1,
                 keepdims=True)

    e2 = jnp.exp(m2 - m1)
    denom = 1.0 + e2
    w1v = 1.0 / denom
    w2v = e2 / denom

    sel = jnp.where(eidx == i1, w1v, 0.0) + jnp.where(eidx == i2, w2v, 0.0)
    pred_ref[...] = jnp.sum(sel * preds, axis=1, keepdims=True)
    rw_ref[...] = jnp.concatenate([w1v, w2v], axis=1)


def kernel(x, W1, b1, W2, b2, S1, s1, S2, s2, We, be):
    n, d = x.shape
    e = S2.shape[1]
    hid1 = W1.shape[1]
    ctxd = W2.shape[1]
    hid2 = S1.shape[1]

    tile = 512
    grid = n // tile
    c1 = lambda i: (0,)
    c2 = lambda i: (0, 0)
    c3 = lambda i: (0, 0, 0)

    preds, rw = pl.pallas_call(
        functools.partial(_body, n_experts=e),
        grid=(grid,),
        in_specs=[
            pl.BlockSpec((tile, d), lambda i: (i, 0)),
            pl.BlockSpec((d, hid1), c2),
            pl.BlockSpec((hid1,), c1),
            pl.BlockSpec((hid1, ctxd), c2),
            pl.BlockSpec((ctxd,), c1),
            pl.BlockSpec((ctxd, hid2), c2),
            pl.BlockSpec((hid2,), c1),
            pl.BlockSpec((hid2, e), c2),
            pl.BlockSpec((e,), c1),
            pl.BlockSpec((e, d, 1), c3),
            pl.BlockSpec((e, 1), c2),
        ],
        out_specs=[
            pl.BlockSpec((tile, 1), lambda i: (i, 0)),
            pl.BlockSpec((tile, TOP_K), lambda i: (i, 0)),
        ],
        out_shape=[
            jax.ShapeDtypeStruct((n, 1), jnp.float32),
            jax.ShapeDtypeStruct((n, TOP_K), jnp.float32),
        ],
        scratch_shapes=[pltpu.VMEM((d, e), jnp.bfloat16)],
    )(x, W1, b1, W2, b2, S1, s1, S2, s2, We, be)
    return (preds, rw)


# speedup vs baseline: 1.2741x; 1.2741x over previous
import functools

import jax
import jax.numpy as jnp
from jax.experimental import pallas as pl
from jax.experimental.pallas import tpu as pltpu

TOP_K = 2
EXPLORATION_BONUS = 0.1


def _body(x_ref, w1_ref, b1_ref, w2_ref, b2_ref, s1_ref, s1b_ref,
          s2_ref, s2b_ref, we_ref, be_ref, pred_ref, rw_ref,
          web_scr, *, n_experts):
    @pl.when(pl.program_id(0) == 0)
    def _prep():
        web_scr[...] = (we_ref[...].reshape(n_experts, -1).T
                        .astype(jnp.bfloat16))

    xt = x_ref[...]
    xb = xt.astype(jnp.bfloat16)
    h = jnp.maximum(
        jnp.dot(xt, w1_ref[...], preferred_element_type=jnp.float32)
        + b1_ref[...].reshape(1, -1), 0.0)
    ctx = jnp.tanh(
        jnp.dot(h, w2_ref[...], preferred_element_type=jnp.float32)
        + b2_ref[...].reshape(1, -1))
    sh = jnp.maximum(
        jnp.dot(ctx, s1_ref[...], preferred_element_type=jnp.float32)
        + s1b_ref[...].reshape(1, -1), 0.0)
    scores = (jnp.dot(sh, s2_ref[...], preferred_element_type=jnp.float32)
              + s2b_ref[...].reshape(1, -1) + EXPLORATION_BONUS)

    preds = (jnp.dot(xb, web_scr[...], preferred_element_type=jnp.float32)
             + be_ref[...].reshape(1, -1))

    scores_t = scores.T
    preds_t = preds.T

    eidx = jax.lax.broadcasted_iota(jnp.int32, scores_t.shape, 0).astype(
        jnp.float32)
    m1 = jnp.max(scores_t, axis=0, keepdims=True)
    i1 = jnp.min(jnp.where(scores_t == m1, eidx, float(n_experts)), axis=0,
                 keepdims=True)
    masked = jnp.where(eidx == i1, -jnp.inf, scores_t)
    m2 = jnp.max(masked, axis=0, keepdims=True)
    i2 = jnp.min(jnp.where(masked == m2, eidx, float(n_experts)), axis=0,
                 keepdims=True)

    e2 = jnp.exp(m2 - m1)
    denom = 1.0 + e2
    w1v = 1.0 / denom
    w2v = e2 / denom

    sel = jnp.where(eidx == i1, w1v, 0.0) + jnp.where(eidx == i2, w2v, 0.0)
    pred_ref[...] = jnp.sum(sel * preds_t, axis=0, keepdims=True)
    rw_ref[...] = jnp.concatenate([w1v, w2v], axis=0)


def kernel(x, W1, b1, W2, b2, S1, s1, S2, s2, We, be):
    n, d = x.shape
    e = S2.shape[1]
    hid1 = W1.shape[1]
    ctxd = W2.shape[1]
    hid2 = S1.shape[1]

    tile = 512
    grid = n // tile
    c1 = lambda i: (0,)
    c2 = lambda i: (0, 0)
    c3 = lambda i: (0, 0, 0)

    preds, rw = pl.pallas_call(
        functools.partial(_body, n_experts=e),
        grid=(grid,),
        in_specs=[
            pl.BlockSpec((tile, d), lambda i: (i, 0)),
            pl.BlockSpec((d, hid1), c2),
            pl.BlockSpec((hid1,), c1),
            pl.BlockSpec((hid1, ctxd), c2),
            pl.BlockSpec((ctxd,), c1),
            pl.BlockSpec((ctxd, hid2), c2),
            pl.BlockSpec((hid2,), c1),
            pl.BlockSpec((hid2, e), c2),
            pl.BlockSpec((e,), c1),
            pl.BlockSpec((e, d, 1), c3),
            pl.BlockSpec((e, 1), c2),
        ],
        out_specs=[
            pl.BlockSpec((1, tile), lambda i: (0, i)),
            pl.BlockSpec((TOP_K, tile), lambda i: (0, i)),
        ],
        out_shape=[
            jax.ShapeDtypeStruct((1, n), jnp.float32),
            jax.ShapeDtypeStruct((TOP_K, n), jnp.float32),
        ],
        scratch_shapes=[pltpu.VMEM((d, e), jnp.bfloat16)],
    )(x, W1, b1, W2, b2, S1, s1, S2, s2, We, be)
    return (preds.reshape(n, 1), rw.T)


# speedup vs baseline: 1.4460x; 1.1349x over previous
import functools

import jax
import jax.numpy as jnp
from jax.experimental import pallas as pl
from jax.experimental.pallas import tpu as pltpu

TOP_K = 2
EXPLORATION_BONUS = 0.1


def _body(x_ref, w1t_ref, b1_ref, w2_ref, b2_ref, s1t_ref, s1b_ref,
          s2t_ref, s2b_ref, we_ref, be_ref, pred_ref, rw_ref,
          w1_scr, s1_scr, s2_scr, web_scr, *, n_experts):
    @pl.when(pl.program_id(0) == 0)
    def _prep():
        w1_scr[...] = w1t_ref[...].T
        s1_scr[...] = s1t_ref[...].T
        s2_scr[...] = s2t_ref[...].T
        web_scr[...] = we_ref[...].T.astype(jnp.bfloat16)

    xt = x_ref[...]
    xb = xt.astype(jnp.bfloat16)
    h = jnp.maximum(
        jnp.dot(xt, w1_scr[...], preferred_element_type=jnp.float32)
        + b1_ref[...].reshape(1, -1), 0.0)
    ctx = jnp.tanh(
        jnp.dot(h, w2_ref[...], preferred_element_type=jnp.float32)
        + b2_ref[...].reshape(1, -1))
    sh = jnp.maximum(
        jnp.dot(ctx, s1_scr[...], preferred_element_type=jnp.float32)
        + s1b_ref[...].reshape(1, -1), 0.0)
    scores = (jnp.dot(sh, s2_scr[...], preferred_element_type=jnp.float32)
              + s2b_ref[...].reshape(1, -1) + EXPLORATION_BONUS)

    preds = (jnp.dot(xb, web_scr[...], preferred_element_type=jnp.float32)
             + be_ref[...].reshape(1, -1))

    scores_t = scores.T
    preds_t = preds.T

    eidx = jax.lax.broadcasted_iota(jnp.int32, scores_t.shape, 0).astype(
        jnp.float32)
    m1 = jnp.max(scores_t, axis=0, keepdims=True)
    i1 = jnp.min(jnp.where(scores_t == m1, eidx, float(n_experts)), axis=0,
                 keepdims=True)
    masked = jnp.where(eidx == i1, -jnp.inf, scores_t)
    m2 = jnp.max(masked, axis=0, keepdims=True)
    i2 = jnp.min(jnp.where(masked == m2, eidx, float(n_experts)), axis=0,
                 keepdims=True)

    e2 = jnp.exp(m2 - m1)
    denom = 1.0 + e2
    w1v = 1.0 / denom
    w2v = e2 / denom

    sel = jnp.where(eidx == i1, w1v, 0.0) + jnp.where(eidx == i2, w2v, 0.0)
    pred_ref[...] = jnp.sum(sel * preds_t, axis=0, keepdims=True)
    rw_ref[...] = jnp.concatenate([w1v, w2v], axis=0)


def kernel(x, W1, b1, W2, b2, S1, s1, S2, s2, We, be):
    n, d = x.shape
    e = S2.shape[1]
    hid1 = W1.shape[1]
    ctxd = W2.shape[1]
    hid2 = S1.shape[1]

    tile = 512
    grid = n // tile
    c1 = lambda i: (0,)
    c2 = lambda i: (0, 0)

    preds, rw = pl.pallas_call(
        functools.partial(_body, n_experts=e),
        grid=(grid,),
        in_specs=[
            pl.BlockSpec((tile, d), lambda i: (i, 0)),
            pl.BlockSpec((hid1, d), c2),
            pl.BlockSpec((hid1,), c1),
            pl.BlockSpec((hid1, ctxd), c2),
            pl.BlockSpec((ctxd,), c1),
            pl.BlockSpec((hid2, ctxd), c2),
            pl.BlockSpec((hid2,), c1),
            pl.BlockSpec((e, hid2), c2),
            pl.BlockSpec((e,), c1),
            pl.BlockSpec((e, d), c2),
            pl.BlockSpec((e,), c1),
        ],
        out_specs=[
            pl.BlockSpec((1, tile), lambda i: (0, i)),
            pl.BlockSpec((TOP_K, tile), lambda i: (0, i)),
        ],
        out_shape=[
            jax.ShapeDtypeStruct((1, n), jnp.float32),
            jax.ShapeDtypeStruct((TOP_K, n), jnp.float32),
        ],
        scratch_shapes=[
            pltpu.VMEM((d, hid1), jnp.float32),
            pltpu.VMEM((ctxd, hid2), jnp.float32),
            pltpu.VMEM((hid2, e), jnp.float32),
            pltpu.VMEM((d, e), jnp.bfloat16),
        ],
    )(x, W1.T, b1, W2, b2, S1.T, s1, S2.T, s2, We[:, :, 0], be.reshape(e))
    return (preds.reshape(n, 1), rw.T)


# speedup vs baseline: 1.8526x; 1.2812x over previous
import functools

import jax
import jax.numpy as jnp
from jax.experimental import pallas as pl
from jax.experimental.pallas import tpu as pltpu

TOP_K = 2
EXPLORATION_BONUS = 0.1


def _body(x_ref, w1t_ref, b1_ref, w2_ref, b2_ref, s1t_ref, s1b_ref,
          s2t_ref, s2b_ref, we_ref, be_ref, pred_ref, rw_ref,
          w1_scr, s1_scr, s2_scr, web_scr, *, n_experts):
    @pl.when(pl.program_id(0) == 0)
    def _prep():
        w1_scr[...] = w1t_ref[...].T
        s1_scr[...] = s1t_ref[...].T
        s2_scr[...] = s2t_ref[...].T
        web_scr[...] = we_ref[...].T.astype(jnp.bfloat16)

    xt = x_ref[...]
    xb = xt.astype(jnp.bfloat16)
    h = jnp.maximum(
        jnp.dot(xt, w1_scr[...], preferred_element_type=jnp.float32)
        + b1_ref[...].reshape(1, -1), 0.0)
    ctx = jnp.tanh(
        jnp.dot(h, w2_ref[...], preferred_element_type=jnp.float32)
        + b2_ref[...].reshape(1, -1))
    sh = jnp.maximum(
        jnp.dot(ctx, s1_scr[...], preferred_element_type=jnp.float32)
        + s1b_ref[...].reshape(1, -1), 0.0)
    scores = (jnp.dot(sh, s2_scr[...], preferred_element_type=jnp.float32)
              + s2b_ref[...].reshape(1, -1) + EXPLORATION_BONUS)

    preds = (jnp.dot(xb, web_scr[...], preferred_element_type=jnp.float32)
             + be_ref[...].reshape(1, -1))

    scores_t = scores.T
    preds_t = preds.T

    eidx = jax.lax.broadcasted_iota(jnp.int32, scores_t.shape, 0).astype(
        jnp.float32)
    m1 = jnp.max(scores_t, axis=0, keepdims=True)
    i1 = jnp.min(jnp.where(scores_t == m1, eidx, float(n_experts)), axis=0,
                 keepdims=True)
    masked = jnp.where(eidx == i1, -jnp.inf, scores_t)
    m2 = jnp.max(masked, axis=0, keepdims=True)
    i2 = jnp.min(jnp.where(masked == m2, eidx, float(n_experts)), axis=0,
                 keepdims=True)

    e2 = jnp.exp(m2 - m1)
    denom = 1.0 + e2
    w1v = 1.0 / denom
    w2v = e2 / denom

    sel = jnp.where(eidx == i1, w1v, 0.0) + jnp.where(eidx == i2, w2v, 0.0)
    pred_ref[...] = jnp.sum(sel * preds_t, axis=0, keepdims=True)
    rw_ref[...] = jnp.concatenate([w1v, w2v], axis=0)


def kernel(x, W1, b1, W2, b2, S1, s1, S2, s2, We, be):
    n, d = x.shape
    e = S2.shape[1]
    hid1 = W1.shape[1]
    ctxd = W2.shape[1]
    hid2 = S1.shape[1]

    tile = 1024
    grid = n // tile
    c1 = lambda i: (0,)
    c2 = lambda i: (0, 0)

    preds, rw = pl.pallas_call(
        functools.partial(_body, n_experts=e),
        grid=(grid,),
        in_specs=[
            pl.BlockSpec((tile, d), lambda i: (i, 0)),
            pl.BlockSpec((hid1, d), c2),
            pl.BlockSpec((hid1,), c1),
            pl.BlockSpec((hid1, ctxd), c2),
            pl.BlockSpec((ctxd,), c1),
            pl.BlockSpec((hid2, ctxd), c2),
            pl.BlockSpec((hid2,), c1),
            pl.BlockSpec((e, hid2), c2),
            pl.BlockSpec((e,), c1),
            pl.BlockSpec((e, d), c2),
            pl.BlockSpec((e,), c1),
        ],
        out_specs=[
            pl.BlockSpec((1, tile), lambda i: (0, i)),
            pl.BlockSpec((TOP_K, tile), lambda i: (0, i)),
        ],
        out_shape=[
            jax.ShapeDtypeStruct((1, n), jnp.float32),
            jax.ShapeDtypeStruct((TOP_K, n), jnp.float32),
        ],
        scratch_shapes=[
            pltpu.VMEM((d, hid1), jnp.float32),
            pltpu.VMEM((ctxd, hid2), jnp.float32),
            pltpu.VMEM((hid2, e), jnp.float32),
            pltpu.VMEM((d, e), jnp.bfloat16),
        ],
    )(x, W1.T, b1, W2, b2, S1.T, s1, S2.T, s2, We[:, :, 0], be.reshape(e))
    return (preds.reshape(n, 1), rw.T)


# speedup vs baseline: 1.9809x; 1.0693x over previous
import functools

import jax
import jax.numpy as jnp
from jax.experimental import pallas as pl
from jax.experimental.pallas import tpu as pltpu

TOP_K = 2
EXPLORATION_BONUS = 0.1


def _body(x_ref, w1t_ref, b1_ref, w2_ref, b2_ref, s1t_ref, s1b_ref,
          s2t_ref, s2b_ref, we_ref, be_ref, pred_ref, rw_ref,
          w1_scr, s1_scr, s2_scr, web_scr, *, n_experts):
    @pl.when(pl.program_id(0) == 0)
    def _prep():
        w1_scr[...] = w1t_ref[...].T
        s1_scr[...] = s1t_ref[...].T
        s2_scr[...] = s2t_ref[...].T
        web_scr[...] = we_ref[...].T.astype(jnp.bfloat16)

    xt = x_ref[...]
    xb = xt.astype(jnp.bfloat16)
    h = jnp.maximum(
        jnp.dot(xt, w1_scr[...], preferred_element_type=jnp.float32)
        + b1_ref[...].reshape(1, -1), 0.0)
    ctx = jnp.tanh(
        jnp.dot(h, w2_ref[...], preferred_element_type=jnp.float32)
        + b2_ref[...].reshape(1, -1))
    sh = jnp.maximum(
        jnp.dot(ctx, s1_scr[...], preferred_element_type=jnp.float32)
        + s1b_ref[...].reshape(1, -1), 0.0)
    scores = (jnp.dot(sh, s2_scr[...], preferred_element_type=jnp.float32)
              + s2b_ref[...].reshape(1, -1) + EXPLORATION_BONUS)

    preds = (jnp.dot(xb, web_scr[...], preferred_element_type=jnp.float32)
             + be_ref[...].reshape(1, -1))

    scores_t = scores.T
    preds_t = preds.T

    eidx = jax.lax.broadcasted_iota(jnp.int32, scores_t.shape, 0).astype(
        jnp.float32)
    m1 = jnp.max(scores_t, axis=0, keepdims=True)
    i1 = jnp.min(jnp.where(scores_t == m1, eidx, float(n_experts)), axis=0,
                 keepdims=True)
    masked = jnp.where(eidx == i1, -jnp.inf, scores_t)
    m2 = jnp.max(masked, axis=0, keepdims=True)
    i2 = jnp.min(jnp.where(masked == m2, eidx, float(n_experts)), axis=0,
                 keepdims=True)

    e2 = jnp.exp(m2 - m1)
    denom = 1.0 + e2
    w1v = 1.0 / denom
    w2v = e2 / denom

    sel = jnp.where(eidx == i1, w1v, 0.0) + jnp.where(eidx == i2, w2v, 0.0)
    pred_ref[...] = jnp.sum(sel * preds_t, axis=0, keepdims=True)
    rw_ref[...] = jnp.concatenate([w1v, w2v], axis=0)


def kernel(x, W1, b1, W2, b2, S1, s1, S2, s2, We, be):
    n, d = x.shape
    e = S2.shape[1]
    hid1 = W1.shape[1]
    ctxd = W2.shape[1]
    hid2 = S1.shape[1]

    tile = 2048
    grid = n // tile
    c1 = lambda i: (0,)
    c2 = lambda i: (0, 0)

    preds, rw = pl.pallas_call(
        functools.partial(_body, n_experts=e),
        grid=(grid,),
        in_specs=[
            pl.BlockSpec((tile, d), lambda i: (i, 0)),
            pl.BlockSpec((hid1, d), c2),
            pl.BlockSpec((hid1,), c1),
            pl.BlockSpec((hid1, ctxd), c2),
            pl.BlockSpec((ctxd,), c1),
            pl.BlockSpec((hid2, ctxd), c2),
            pl.BlockSpec((hid2,), c1),
            pl.BlockSpec((e, hid2), c2),
            pl.BlockSpec((e,), c1),
            pl.BlockSpec((e, d), c2),
            pl.BlockSpec((e,), c1),
        ],
        out_specs=[
            pl.BlockSpec((1, tile), lambda i: (0, i)),
            pl.BlockSpec((TOP_K, tile), lambda i: (0, i)),
        ],
        out_shape=[
            jax.ShapeDtypeStruct((1, n), jnp.float32),
            jax.ShapeDtypeStruct((TOP_K, n), jnp.float32),
        ],
        scratch_shapes=[
            pltpu.VMEM((d, hid1), jnp.float32),
            pltpu.VMEM((ctxd, hid2), jnp.float32),
            pltpu.VMEM((hid2, e), jnp.float32),
            pltpu.VMEM((d, e), jnp.bfloat16),
        ],
    )(x, W1.T, b1, W2, b2, S1.T, s1, S2.T, s2, We[:, :, 0], be.reshape(e))
    return (preds.reshape(n, 1), rw.T)


# speedup vs baseline: 2.0266x; 1.0230x over previous
import functools

import jax
import jax.numpy as jnp
from jax.experimental import pallas as pl
from jax.experimental.pallas import tpu as pltpu

TOP_K = 2
EXPLORATION_BONUS = 0.1


def _body(x_ref, w1t_ref, b1_ref, w2_ref, b2_ref, s1t_ref, s1b_ref,
          s2t_ref, s2b_ref, we_ref, be_ref, pred_ref, rw_ref,
          w1_scr, s1_scr, s2_scr, web_scr, *, n_experts):
    @pl.when(pl.program_id(0) == 0)
    def _prep():
        w1_scr[...] = w1t_ref[...].T
        s1_scr[...] = s1t_ref[...].T
        s2_scr[...] = s2t_ref[...].T
        web_scr[...] = we_ref[...].T.astype(jnp.bfloat16)

    xt = x_ref[...]
    xb = xt.astype(jnp.bfloat16)
    h = jnp.maximum(
        jnp.dot(xt, w1_scr[...], preferred_element_type=jnp.float32)
        + b1_ref[...].reshape(1, -1), 0.0)
    ctx = jnp.tanh(
        jnp.dot(h, w2_ref[...], preferred_element_type=jnp.float32)
        + b2_ref[...].reshape(1, -1))
    sh = jnp.maximum(
        jnp.dot(ctx, s1_scr[...], preferred_element_type=jnp.float32)
        + s1b_ref[...].reshape(1, -1), 0.0)
    scores = (jnp.dot(sh, s2_scr[...], preferred_element_type=jnp.float32)
              + s2b_ref[...].reshape(1, -1) + EXPLORATION_BONUS)

    preds = (jnp.dot(xb, web_scr[...], preferred_element_type=jnp.float32)
             + be_ref[...].reshape(1, -1))

    scores_t = scores.T
    preds_t = preds.T

    eidx = jax.lax.broadcasted_iota(jnp.int32, scores_t.shape, 0).astype(
        jnp.float32)
    m1 = jnp.max(scores_t, axis=0, keepdims=True)
    i1 = jnp.min(jnp.where(scores_t == m1, eidx, float(n_experts)), axis=0,
                 keepdims=True)
    masked = jnp.where(eidx == i1, -jnp.inf, scores_t)
    m2 = jnp.max(masked, axis=0, keepdims=True)
    i2 = jnp.min(jnp.where(masked == m2, eidx, float(n_experts)), axis=0,
                 keepdims=True)

    e2 = jnp.exp(m2 - m1)
    denom = 1.0 + e2
    w1v = 1.0 / denom
    w2v = e2 / denom

    sel = jnp.where(eidx == i1, w1v, 0.0) + jnp.where(eidx == i2, w2v, 0.0)
    pred_ref[...] = jnp.sum(sel * preds_t, axis=0, keepdims=True)
    rw_ref[...] = jnp.concatenate([w1v, w2v], axis=0)


def kernel(x, W1, b1, W2, b2, S1, s1, S2, s2, We, be):
    n, d = x.shape
    e = S2.shape[1]
    hid1 = W1.shape[1]
    ctxd = W2.shape[1]
    hid2 = S1.shape[1]

    tile = 4096
    grid = n // tile
    c1 = lambda i: (0,)
    c2 = lambda i: (0, 0)

    preds, rw = pl.pallas_call(
        functools.partial(_body, n_experts=e),
        grid=(grid,),
        in_specs=[
            pl.BlockSpec((tile, d), lambda i: (i, 0)),
            pl.BlockSpec((hid1, d), c2),
            pl.BlockSpec((hid1,), c1),
            pl.BlockSpec((hid1, ctxd), c2),
            pl.BlockSpec((ctxd,), c1),
            pl.BlockSpec((hid2, ctxd), c2),
            pl.BlockSpec((hid2,), c1),
            pl.BlockSpec((e, hid2), c2),
            pl.BlockSpec((e,), c1),
            pl.BlockSpec((e, d), c2),
            pl.BlockSpec((e,), c1),
        ],
        out_specs=[
            pl.BlockSpec((1, tile), lambda i: (0, i)),
            pl.BlockSpec((TOP_K, tile), lambda i: (0, i)),
        ],
        out_shape=[
            jax.ShapeDtypeStruct((1, n), jnp.float32),
            jax.ShapeDtypeStruct((TOP_K, n), jnp.float32),
        ],
        scratch_shapes=[
            pltpu.VMEM((d, hid1), jnp.float32),
            pltpu.VMEM((ctxd, hid2), jnp.float32),
            pltpu.VMEM((hid2, e), jnp.float32),
            pltpu.VMEM((d, e), jnp.bfloat16),
        ],
    )(x, W1.T, b1, W2, b2, S1.T, s1, S2.T, s2, We[:, :, 0], be.reshape(e))
    return (preds.reshape(n, 1), rw.T)
